# baseline (speedup 1.0000x reference)
import functools
import jax
import jax.numpy as jnp
from jax import lax
from jax.experimental import pallas as pl
from jax.experimental.pallas import tpu as pltpu
from jax.experimental.pallas import tpu_sc as plsc

B = 16384
F = 64
K = int(0.3 * B)
BLK = 8192
GRID = B // BLK

NT = 16
N = B // NT
NCH = N // 16
HSZ = 384
DUMP = jnp.int32(256)
CELLS = 258 * 128
CZ = CELLS // NT


def _tc_body(x_ref, t_ref, out_ref, key_ref):
    d = x_ref[...] - t_ref[...]
    part = jnp.sum(d * d, axis=1) * (1.0 / F)
    out_ref[...] = part
    key_ref[...] = jax.lax.bitcast_convert_type(part, jnp.int32)


_mesh = plsc.VectorSubcoreMesh(core_axis_name="c", subcore_axis_name="s", num_cores=1)


@functools.partial(
    pl.kernel,
    mesh=_mesh,
    out_type=(
        jax.ShapeDtypeStruct((16,), jnp.float32),
        jax.ShapeDtypeStruct((16,), jnp.int32),
    ),
    scratch_types=[
        pltpu.VMEM((N,), jnp.float32),
        pltpu.VMEM((N,), jnp.int32),
        pltpu.VMEM((8, 128), jnp.int32),
        pltpu.VMEM((8, 128), jnp.int32),
        pltpu.VMEM((CZ,), jnp.int32),
        pltpu.VMEM((HSZ,), jnp.int32),
        pltpu.VMEM((16,), jnp.int32),
        pltpu.VMEM((16,), jnp.int32),
        pltpu.VMEM((2048,), jnp.int32),
        pltpu.VMEM((16,), jnp.float32),
        pltpu.VMEM((16, 16), jnp.float32),
        pltpu.VMEM_SHARED((CELLS,), jnp.int32),
        pltpu.VMEM_SHARED((CELLS,), jnp.int32),
        pltpu.VMEM_SHARED((CELLS,), jnp.int32),
        pltpu.VMEM_SHARED((CELLS,), jnp.int32),
        pltpu.VMEM_SHARED((256,), jnp.int32),
        pltpu.VMEM_SHARED((256,), jnp.int32),
        pltpu.VMEM_SHARED((256,), jnp.int32),
        pltpu.VMEM_SHARED((256,), jnp.int32),
        pltpu.VMEM_SHARED((16,), jnp.int32),
        pltpu.VMEM_SHARED((16,), jnp.int32),
        pltpu.VMEM_SHARED((16,), jnp.int32),
        pltpu.VMEM_SHARED((16,), jnp.int32),
        pltpu.VMEM_SHARED((16, 16), jnp.float32),
    ],
)
def _sc_topk(
    loss_hbm,
    key_hbm,
    outf_hbm,
    outi_hbm,
    lossv,
    keyv,
    idx2d,
    ones2d,
    zv,
    histv,
    ctrlv,
    tmpv,
    cellv,
    resv,
    sumv,
    h0,
    h1,
    h2,
    h3,
    hc0,
    hc1,
    hc2,
    hc3,
    cs0,
    cs1,
    cs2,
    cs3,
    sums,
):
    tid = lax.axis_index("s")
    H = [h0, h1, h2, h3]
    HC = [hc0, hc1, hc2, hc3]
    CS = [cs0, cs1, cs2, cs3]
    iota16 = lax.iota(jnp.int32, 16)
    zero16i = jnp.zeros((16,), jnp.int32)
    zero16f = jnp.zeros((16,), jnp.float32)

    for c in range(CZ // 16):
        zv[pl.ds(c * 16, 16)] = zero16i
    for j in range(8):
        for c in range(8):
            ones2d[j, pl.ds(c * 16, 16)] = zero16i + 1

    pltpu.sync_copy(loss_hbm.at[pl.ds(tid * N, N)], lossv)
    pltpu.sync_copy(key_hbm.at[pl.ds(tid * N, N)], keyv)

    for r in range(4):
        pltpu.sync_copy(zv, H[r].at[pl.ds(tid * CZ, CZ)])
    plsc.subcore_barrier()

    p = jnp.int32(0)
    c_above = jnp.int32(0)
    need = jnp.int32(K)
    SH = [23, 15, 7, 0]
    PSH = [0, 23, 15, 7]
    MSK = [jnp.int32(255), jnp.int32(255), jnp.int32(255), jnp.int32(127)]
    NV = [16, 16, 16, 8]

    for r in range(4):
        for c in range(NCH):
            kv = keyv[pl.ds(c * 16, 16)]
            bv = lax.bitwise_and(lax.shift_right_logical(kv, SH[r]), MSK[r])
            if r > 0:
                pref = lax.shift_right_logical(kv, PSH[r])
                bv = jnp.where(pref == p, bv, DUMP)
            cell = bv * 128 + ((c % 8) * 16 + iota16)
            idx2d[c // 8, pl.ds((c % 8) * 16, 16)] = cell
        for j in range(8):
            pltpu.sync_copy(ones2d.at[j], H[r].at[idx2d.at[j]], add=True)
        plsc.subcore_barrier()

        pltpu.sync_copy(H[r].at[pl.ds(tid * 2048, 2048)], cellv)
        tot_vec = zero16i
        for bi in range(16):
            sv = cellv[pl.ds(bi * 128, 16)]
            for v8 in range(1, 8):
                sv = sv + cellv[pl.ds(bi * 128 + v8 * 16, 16)]
            tot_b = jnp.int32(0)
            for l in range(16):
                tot_b = tot_b + sv[l]
            tot_vec = tot_vec + jnp.where(iota16 == bi, tot_b, 0)
        tmpv[...] = tot_vec
        pltpu.sync_copy(tmpv, HC[r].at[pl.ds(tid * 16, 16)])
        plsc.subcore_barrier()

        @pl.when(tid == 0)
        def _():
            pltpu.sync_copy(HC[r], histv.at[pl.ds(0, 256)])
            nv = NV[r]

            def step(j, carry):
                suf, b_sel, c_sel = carry
                vi = nv - 1 - j
                hv = histv[pl.ds(vi * 16, 16)]
                for l in range(15, -1, -1):
                    v = hv[l]
                    suf_new = suf + v
                    ci = ((suf < need) & (suf_new >= need)).astype(jnp.int32)
                    b_sel = b_sel + ci * (vi * 16 + l)
                    c_sel = c_sel + ci * suf
                    suf = suf_new
                return (suf, b_sel, c_sel)

            _, b_r, c_ab_sel = lax.fori_loop(
                0, nv, step, (jnp.int32(0), jnp.int32(0), jnp.int32(0))
            )
            c_above_new = c_ab_sel + c_above
            need_new = K - c_above_new
            if r < 3:
                p_new = lax.shift_left(p, 8) | b_r
            else:
                p_new = lax.shift_left(p, 7) | b_r
            ctrlv[...] = (
                jnp.where(iota16 == 0, p_new, 0)
                + jnp.where(iota16 == 1, c_above_new, 0)
                + jnp.where(iota16 == 2, need_new, 0)
            )
            pltpu.sync_copy(ctrlv, CS[r])

        plsc.subcore_barrier()
        pltpu.sync_copy(CS[r], ctrlv)
        cv = ctrlv[...]
        p = cv[0]
        c_above = cv[1]
        need = cv[2]

    t = p
    m = need

    acc = zero16f
    for c in range(NCH):
        kv = keyv[pl.ds(c * 16, 16)]
        lv = lossv[pl.ds(c * 16, 16)]
        acc = acc + jnp.where(kv > t, lv, jnp.float32(0.0))
    resv[...] = acc
    pltpu.sync_copy(resv, sums.at[tid])
    plsc.subcore_barrier()

    @pl.when(tid == 0)
    def _():
        pltpu.sync_copy(sums, sumv)
        tot = zero16f
        for i in range(16):
            tot = tot + sumv[i, pl.ds(0, 16)]
        s_tot = jnp.float32(0.0)
        for l in range(16):
            s_tot = s_tot + tot[l]
        resv[...] = jnp.where(iota16 == 0, s_tot, jnp.float32(0.0))
        pltpu.sync_copy(resv, outf_hbm)
        ctrlv[...] = jnp.where(iota16 == 0, t, 0) + jnp.where(iota16 == 1, m, 0)
        pltpu.sync_copy(ctrlv, outi_hbm)


def kernel(input, target):
    loss, keys = pl.pallas_call(
        _tc_body,
        grid=(GRID,),
        in_specs=[
            pl.BlockSpec((BLK, F), lambda i: (i, 0)),
            pl.BlockSpec((BLK, F), lambda i: (i, 0)),
        ],
        out_specs=[
            pl.BlockSpec((BLK,), lambda i: (i,)),
            pl.BlockSpec((BLK,), lambda i: (i,)),
        ],
        out_shape=[
            jax.ShapeDtypeStruct((B,), jnp.float32),
            jax.ShapeDtypeStruct((B,), jnp.int32),
        ],
    )(input, target)
    s, ti = _sc_topk(loss, keys)
    tf = jax.lax.bitcast_convert_type(ti[0], jnp.float32)
    mf = ti[1].astype(jnp.float32)
    return (s[0] + mf * tf) * (1.0 / K)


# speedup vs baseline: 1.0439x; 1.0439x over previous
import functools
import jax
import jax.numpy as jnp
from jax import lax
from jax.experimental import pallas as pl
from jax.experimental.pallas import tpu as pltpu
from jax.experimental.pallas import tpu_sc as plsc

B = 16384
F = 64
K = int(0.3 * B)
BLK = 8192
GRID = B // BLK

NT = 16
N = B // NT
NCH = N // 16
HSZ = 384
DUMP = jnp.int32(256)
CELLS = 258 * 128
CZ = CELLS // NT


def _tc_body(x_ref, t_ref, out_ref, key_ref):
    d = x_ref[...] - t_ref[...]
    part = jnp.sum(d * d, axis=1) * (1.0 / F)
    out_ref[...] = part
    key_ref[...] = jax.lax.bitcast_convert_type(part, jnp.int32)


_mesh = plsc.VectorSubcoreMesh(core_axis_name="c", subcore_axis_name="s", num_cores=1)


@functools.partial(
    pl.kernel,
    mesh=_mesh,
    out_type=(
        jax.ShapeDtypeStruct((16,), jnp.float32),
        jax.ShapeDtypeStruct((16,), jnp.int32),
    ),
    scratch_types=[
        pltpu.VMEM((N,), jnp.float32),
        pltpu.VMEM((N,), jnp.int32),
        pltpu.VMEM((8, 128), jnp.int32),
        pltpu.VMEM((8, 128), jnp.int32),
        pltpu.VMEM((CZ,), jnp.int32),
        pltpu.VMEM((HSZ,), jnp.int32),
        pltpu.VMEM((16,), jnp.int32),
        pltpu.VMEM((16,), jnp.int32),
        pltpu.VMEM((2048,), jnp.int32),
        pltpu.VMEM((16,), jnp.float32),
        pltpu.VMEM((16, 16), jnp.float32),
        pltpu.VMEM_SHARED((CELLS,), jnp.int32),
        pltpu.VMEM_SHARED((CELLS,), jnp.int32),
        pltpu.VMEM_SHARED((CELLS,), jnp.int32),
        pltpu.VMEM_SHARED((CELLS,), jnp.int32),
        pltpu.VMEM_SHARED((256,), jnp.int32),
        pltpu.VMEM_SHARED((256,), jnp.int32),
        pltpu.VMEM_SHARED((256,), jnp.int32),
        pltpu.VMEM_SHARED((256,), jnp.int32),
        pltpu.VMEM_SHARED((16,), jnp.int32),
        pltpu.VMEM_SHARED((16,), jnp.int32),
        pltpu.VMEM_SHARED((16,), jnp.int32),
        pltpu.VMEM_SHARED((16,), jnp.int32),
        pltpu.VMEM_SHARED((16, 16), jnp.float32),
        pltpu.SemaphoreType.DMA((8,)),
    ],
)
def _sc_topk(
    loss_hbm,
    key_hbm,
    outf_hbm,
    outi_hbm,
    lossv,
    keyv,
    idx2d,
    ones2d,
    zv,
    histv,
    ctrlv,
    tmpv,
    cellv,
    resv,
    sumv,
    h0,
    h1,
    h2,
    h3,
    hc0,
    hc1,
    hc2,
    hc3,
    cs0,
    cs1,
    cs2,
    cs3,
    sums,
    dsem,
):
    tid = lax.axis_index("s")
    H = [h0, h1, h2, h3]
    HC = [hc0, hc1, hc2, hc3]
    CS = [cs0, cs1, cs2, cs3]
    iota16 = lax.iota(jnp.int32, 16)
    zero16i = jnp.zeros((16,), jnp.int32)
    zero16f = jnp.zeros((16,), jnp.float32)

    for c in range(CZ // 16):
        zv[pl.ds(c * 16, 16)] = zero16i
    for j in range(8):
        for c in range(8):
            ones2d[j, pl.ds(c * 16, 16)] = zero16i + 1

    cps = [
        pltpu.async_copy(loss_hbm.at[pl.ds(tid * N, N)], lossv, dsem.at[0]),
        pltpu.async_copy(key_hbm.at[pl.ds(tid * N, N)], keyv, dsem.at[1]),
    ]
    for r in range(4):
        cps.append(
            pltpu.async_copy(zv, H[r].at[pl.ds(tid * CZ, CZ)], dsem.at[2 + r])
        )
    for cp in cps:
        cp.wait()
    plsc.subcore_barrier()

    p = jnp.int32(0)
    c_above = jnp.int32(0)
    need = jnp.int32(K)
    SH = [23, 15, 7, 0]
    PSH = [0, 23, 15, 7]
    MSK = [jnp.int32(255), jnp.int32(255), jnp.int32(255), jnp.int32(127)]
    NV = [16, 16, 16, 8]

    for r in range(4):
        for c in range(NCH):
            kv = keyv[pl.ds(c * 16, 16)]
            bv = lax.bitwise_and(lax.shift_right_logical(kv, SH[r]), MSK[r])
            if r > 0:
                pref = lax.shift_right_logical(kv, PSH[r])
                bv = jnp.where(pref == p, bv, DUMP)
            cell = bv * 128 + ((c % 8) * 16 + iota16)
            idx2d[c // 8, pl.ds((c % 8) * 16, 16)] = cell
        scps = [
            pltpu.async_copy(
                ones2d.at[j], H[r].at[idx2d.at[j]], dsem.at[j], add=True
            )
            for j in range(8)
        ]
        for cp in scps:
            cp.wait()
        plsc.subcore_barrier()

        pltpu.sync_copy(H[r].at[pl.ds(tid * 2048, 2048)], cellv)
        tot_vec = zero16i
        for bi in range(16):
            sv = cellv[pl.ds(bi * 128, 16)]
            for v8 in range(1, 8):
                sv = sv + cellv[pl.ds(bi * 128 + v8 * 16, 16)]
            tot_b = jnp.int32(0)
            for l in range(16):
                tot_b = tot_b + sv[l]
            tot_vec = tot_vec + jnp.where(iota16 == bi, tot_b, 0)
        tmpv[...] = tot_vec
        pltpu.sync_copy(tmpv, HC[r].at[pl.ds(tid * 16, 16)])
        plsc.subcore_barrier()

        @pl.when(tid == 0)
        def _():
            pltpu.sync_copy(HC[r], histv.at[pl.ds(0, 256)])
            nv = NV[r]

            def step(j, carry):
                suf, b_sel, c_sel = carry
                vi = nv - 1 - j
                hv = histv[pl.ds(vi * 16, 16)]
                for l in range(15, -1, -1):
                    v = hv[l]
                    suf_new = suf + v
                    ci = ((suf < need) & (suf_new >= need)).astype(jnp.int32)
                    b_sel = b_sel + ci * (vi * 16 + l)
                    c_sel = c_sel + ci * suf
                    suf = suf_new
                return (suf, b_sel, c_sel)

            _, b_r, c_ab_sel = lax.fori_loop(
                0, nv, step, (jnp.int32(0), jnp.int32(0), jnp.int32(0))
            )
            c_above_new = c_ab_sel + c_above
            need_new = K - c_above_new
            if r < 3:
                p_new = lax.shift_left(p, 8) | b_r
            else:
                p_new = lax.shift_left(p, 7) | b_r
            ctrlv[...] = (
                jnp.where(iota16 == 0, p_new, 0)
                + jnp.where(iota16 == 1, c_above_new, 0)
                + jnp.where(iota16 == 2, need_new, 0)
            )
            pltpu.sync_copy(ctrlv, CS[r])

        plsc.subcore_barrier()
        pltpu.sync_copy(CS[r], ctrlv)
        cv = ctrlv[...]
        p = cv[0]
        c_above = cv[1]
        need = cv[2]

    t = p
    m = need

    acc = zero16f
    for c in range(NCH):
        kv = keyv[pl.ds(c * 16, 16)]
        lv = lossv[pl.ds(c * 16, 16)]
        acc = acc + jnp.where(kv > t, lv, jnp.float32(0.0))
    resv[...] = acc
    pltpu.sync_copy(resv, sums.at[tid])
    plsc.subcore_barrier()

    @pl.when(tid == 0)
    def _():
        pltpu.sync_copy(sums, sumv)
        tot = zero16f
        for i in range(16):
            tot = tot + sumv[i, pl.ds(0, 16)]
        s_tot = jnp.float32(0.0)
        for l in range(16):
            s_tot = s_tot + tot[l]
        resv[...] = jnp.where(iota16 == 0, s_tot, jnp.float32(0.0))
        pltpu.sync_copy(resv, outf_hbm)
        ctrlv[...] = jnp.where(iota16 == 0, t, 0) + jnp.where(iota16 == 1, m, 0)
        pltpu.sync_copy(ctrlv, outi_hbm)


def kernel(input, target):
    loss, keys = pl.pallas_call(
        _tc_body,
        grid=(GRID,),
        in_specs=[
            pl.BlockSpec((BLK, F), lambda i: (i, 0)),
            pl.BlockSpec((BLK, F), lambda i: (i, 0)),
        ],
        out_specs=[
            pl.BlockSpec((BLK,), lambda i: (i,)),
            pl.BlockSpec((BLK,), lambda i: (i,)),
        ],
        out_shape=[
            jax.ShapeDtypeStruct((B,), jnp.float32),
            jax.ShapeDtypeStruct((B,), jnp.int32),
        ],
    )(input, target)
    s, ti = _sc_topk(loss, keys)
    tf = jax.lax.bitcast_convert_type(ti[0], jnp.float32)
    mf = ti[1].astype(jnp.float32)
    return (s[0] + mf * tf) * (1.0 / K)


# speedup vs baseline: 2.9872x; 2.8617x over previous
import jax
import jax.numpy as jnp
from jax.experimental import pallas as pl
from jax.experimental.pallas import tpu as pltpu

B = 16384
F = 64
K = int(0.3 * B)
BLK = 8192
GRID = B // BLK


def _body(x_ref, t_ref, out_ref, loss_ref):
    i = pl.program_id(0)
    d = x_ref[...] - t_ref[...]
    part = jnp.sum(d * d, axis=1) * (1.0 / F)
    loss_ref[pl.ds(i * (BLK // 128), BLK // 128), :] = part.reshape(BLK // 128, 128)

    @pl.when(i == GRID - 1)
    def _():
        loss = loss_ref[...]
        keys = jax.lax.bitcast_convert_type(loss, jnp.int32)

        t = jnp.int32(0)
        for g in range(8):
            sh = 28 - 4 * g
            chosen = jnp.int32(0)
            for n in range(1, 8 if g == 0 else 16):
                cand = t | (n << sh)
                cnt = jnp.sum((keys >= cand).astype(jnp.int32))
                chosen = chosen + (cnt >= K).astype(jnp.int32)
            t = t | (chosen << sh)

        gt = keys > t
        c_gt = jnp.sum(gt.astype(jnp.int32))
        s_gt = jnp.sum(jnp.where(gt, loss, 0.0))
        tf = jax.lax.bitcast_convert_type(t, jnp.float32)
        out_ref[0] = (s_gt + (K - c_gt).astype(jnp.float32) * tf) * (1.0 / K)


def kernel(input, target):
    res = pl.pallas_call(
        _body,
        grid=(GRID,),
        in_specs=[
            pl.BlockSpec((BLK, F), lambda i: (i, 0)),
            pl.BlockSpec((BLK, F), lambda i: (i, 0)),
        ],
        out_specs=pl.BlockSpec(memory_space=pltpu.SMEM),
        out_shape=jax.ShapeDtypeStruct((1,), jnp.float32),
        scratch_shapes=[pltpu.VMEM((128, 128), jnp.float32)],
    )(input, target)
    return res[0]
